# Initial kernel scaffold; baseline (speedup 1.0000x reference)
import jax
import jax.numpy as jnp
from jax.experimental import pallas as pl


def kernel(mel, key_bins):
    raise NotImplementedError("write your pallas kernel here")


# speedup vs baseline: 4.8384x; 4.8384x over previous
import functools

import jax
import jax.numpy as jnp
import numpy as np
from jax.experimental import pallas as pl

_IDX = np.float32(np.float32(np.float32(85.0) / np.float32(100.0)) * np.float32(87.0))
_P = np.float32(_IDX - np.float32(73.0))
_Q = np.float32(np.float32(1.0) - _P)

_NUM_DISTINCT = 15


def _body(g1_ref, g2_ref, g3_ref, z2_ref, z3_ref, mel_ref, out_ref):
    spec = jnp.exp(mel_ref[0])
    dot = functools.partial(
        jax.lax.dot_general,
        dimension_numbers=(((1,), (0,)), ((), ())),
        precision=jax.lax.Precision.HIGHEST,
        preferred_element_type=jnp.float32,
    )
    g1 = dot(g1_ref[...], spec)
    g2 = dot(g2_ref[...], spec)
    g3 = dot(g3_ref[...], spec)
    e = (g1 * (g2 + z2_ref[...])) * (g3 + z3_ref[...])

    tcols = e.shape[1]
    neg = jnp.float32(-jnp.inf)
    v = jnp.full((1, tcols), jnp.inf, jnp.float32)
    cprev = jnp.zeros((1, tcols), jnp.float32)
    s74 = jnp.zeros((1, tcols), jnp.float32)
    s73 = jnp.zeros((1, tcols), jnp.float32)
    for _ in range(_NUM_DISTINCT):
        masked = jnp.where(e < v, e, neg)
        nxt = jnp.max(masked, axis=0, keepdims=True)
        c = jnp.sum(
            jnp.where(e >= nxt, jnp.float32(1.0), jnp.float32(0.0)),
            axis=0,
            keepdims=True,
        )
        s74 = jnp.where((cprev < 14.0) & (c >= 14.0), nxt, s74)
        s73 = jnp.where((cprev < 15.0) & (c >= 15.0), nxt, s73)
        v = nxt
        cprev = c
    thresh = (s73 * _Q) + (s74 * _P)
    out_ref[0] = jnp.where(e >= thresh, jnp.float32(1.0), jnp.float32(0.0))


def kernel(mel, key_bins):
    if mel.ndim == 4:
        mel = mel[:, 0]
    b, m, t = mel.shape
    k = key_bins.shape[0]
    kb = key_bins.astype(jnp.int32)
    g1 = jax.nn.one_hot(kb, m, dtype=jnp.float32)
    g2 = jax.nn.one_hot(2 * kb, m, dtype=jnp.float32)
    g3 = jax.nn.one_hot(3 * kb, m, dtype=jnp.float32)
    z2 = jnp.where(2 * kb < m, 0.0, 1.0).astype(jnp.float32)[:, None]
    z3 = jnp.where(3 * kb < m, 0.0, 1.0).astype(jnp.float32)[:, None]
    tb = 512
    grid = (b, t // tb)
    out = pl.pallas_call(
        _body,
        grid=grid,
        in_specs=[
            pl.BlockSpec((k, m), lambda bi, ti: (0, 0)),
            pl.BlockSpec((k, m), lambda bi, ti: (0, 0)),
            pl.BlockSpec((k, m), lambda bi, ti: (0, 0)),
            pl.BlockSpec((k, 1), lambda bi, ti: (0, 0)),
            pl.BlockSpec((k, 1), lambda bi, ti: (0, 0)),
            pl.BlockSpec((1, m, tb), lambda bi, ti: (bi, 0, ti)),
        ],
        out_specs=pl.BlockSpec((1, k, tb), lambda bi, ti: (bi, 0, ti)),
        out_shape=jax.ShapeDtypeStruct((b, k, t), jnp.float32),
    )(g1, g2, g3, z2, z3, mel)
    return (out, out)


# speedup vs baseline: 5.3701x; 1.1099x over previous
import functools

import jax
import jax.numpy as jnp
import numpy as np
from jax.experimental import pallas as pl

_IDX = np.float32(np.float32(np.float32(85.0) / np.float32(100.0)) * np.float32(87.0))
_P = np.float32(_IDX - np.float32(73.0))
_Q = np.float32(np.float32(1.0) - _P)

_NUM_DISTINCT = 15
_D = 64


def _body(gcat_ref, z2_ref, z3_ref, w_ref, p_ref, mel_ref, out_ref):
    spec = jnp.exp(mel_ref[0])
    g = jax.lax.dot_general(
        gcat_ref[...],
        spec,
        dimension_numbers=(((1,), (0,)), ((), ())),
        precision=jax.lax.Precision.HIGHEST,
        preferred_element_type=jnp.float32,
    )
    g1 = g[:_D]
    g2 = g[_D:2 * _D]
    g3 = g[2 * _D:]
    e = (g1 * (g2 + z2_ref[...])) * (g3 + z3_ref[...])

    tcols = e.shape[1]
    neg = jnp.float32(-jnp.inf)
    w = w_ref[...]
    consumed = jnp.zeros(e.shape, jnp.bool_)
    cprev = jnp.zeros((1, tcols), jnp.float32)
    s74 = jnp.zeros((1, tcols), jnp.float32)
    s73 = jnp.zeros((1, tcols), jnp.float32)
    for _ in range(_NUM_DISTINCT):
        masked = jnp.where(consumed, neg, e)
        nxt = jnp.max(masked, axis=0, keepdims=True)
        consumed = e >= nxt
        c = jnp.sum(
            jnp.where(consumed, w, jnp.float32(0.0)),
            axis=0,
            keepdims=True,
        )
        s74 = jnp.where((cprev < 14.0) & (c >= 14.0), nxt, s74)
        s73 = jnp.where((cprev < 15.0) & (c >= 15.0), nxt, s73)
        cprev = c
    thresh = (s73 * _Q) + (s74 * _P)
    cmp01 = jnp.where(e >= thresh, jnp.float32(1.0), jnp.float32(0.0)).astype(
        jnp.bfloat16
    )
    out_ref[0] = jax.lax.dot_general(
        p_ref[...],
        cmp01,
        dimension_numbers=(((1,), (0,)), ((), ())),
        preferred_element_type=jnp.float32,
    )


def kernel(mel, key_bins):
    if mel.ndim == 4:
        mel = mel[:, 0]
    b, m, t = mel.shape
    k = key_bins.shape[0]
    kb = key_bins.astype(jnp.int32)
    u = jnp.unique(kb, size=_D, fill_value=-1)
    pmat = (u[None, :] == kb[:, None]).astype(jnp.bfloat16)
    w = jnp.sum((u[None, :] == kb[:, None]).astype(jnp.float32), axis=0)[:, None]
    g1 = jax.nn.one_hot(u, m, dtype=jnp.float32)
    g2 = jax.nn.one_hot(2 * u, m, dtype=jnp.float32)
    g3 = jax.nn.one_hot(3 * u, m, dtype=jnp.float32)
    gcat = jnp.concatenate([g1, g2, g3], axis=0)
    z2 = jnp.where(2 * u < m, 0.0, 1.0).astype(jnp.float32)[:, None]
    z3 = jnp.where(3 * u < m, 0.0, 1.0).astype(jnp.float32)[:, None]
    tb = 512
    grid = (b, t // tb)
    out = pl.pallas_call(
        _body,
        grid=grid,
        in_specs=[
            pl.BlockSpec((3 * _D, m), lambda bi, ti: (0, 0)),
            pl.BlockSpec((_D, 1), lambda bi, ti: (0, 0)),
            pl.BlockSpec((_D, 1), lambda bi, ti: (0, 0)),
            pl.BlockSpec((_D, 1), lambda bi, ti: (0, 0)),
            pl.BlockSpec((k, _D), lambda bi, ti: (0, 0)),
            pl.BlockSpec((1, m, tb), lambda bi, ti: (bi, 0, ti)),
        ],
        out_specs=pl.BlockSpec((1, k, tb), lambda bi, ti: (bi, 0, ti)),
        out_shape=jax.ShapeDtypeStruct((b, k, t), jnp.float32),
    )(gcat, z2, z3, w, pmat, mel)
    return (out, out)


# speedup vs baseline: 7.0340x; 1.3098x over previous
import functools

import jax
import jax.numpy as jnp
import numpy as np
from jax.experimental import pallas as pl

_IDX = np.float32(np.float32(np.float32(85.0) / np.float32(100.0)) * np.float32(87.0))
_P = np.float32(_IDX - np.float32(73.0))
_Q = np.float32(np.float32(1.0) - _P)

_NUM_DISTINCT = 15
_D = 64


def _body(gcat_ref, z2_ref, z3_ref, w_ref, p_ref, mel_ref, out_ref):
    spec = jnp.exp(mel_ref[0])
    g = jax.lax.dot_general(
        gcat_ref[...],
        spec,
        dimension_numbers=(((1,), (0,)), ((), ())),
        precision=jax.lax.Precision.HIGHEST,
        preferred_element_type=jnp.float32,
    )
    g1 = g[:_D]
    g2 = g[_D:2 * _D]
    g3 = g[2 * _D:]
    e = (g1 * (g2 + z2_ref[...])) * (g3 + z3_ref[...])

    tcols = e.shape[1]
    neg = jnp.float32(-jnp.inf)
    w = w_ref[...]
    consumed = jnp.zeros(e.shape, jnp.bool_)
    cprev = jnp.zeros((1, tcols), jnp.float32)
    s74 = jnp.zeros((1, tcols), jnp.float32)
    s73 = jnp.zeros((1, tcols), jnp.float32)
    for _ in range(_NUM_DISTINCT):
        masked = jnp.where(consumed, neg, e)
        nxt = jnp.max(masked, axis=0, keepdims=True)
        consumed = e >= nxt
        c = jnp.sum(
            jnp.where(consumed, w, jnp.float32(0.0)),
            axis=0,
            keepdims=True,
        )
        s74 = jnp.where((cprev < 14.0) & (c >= 14.0), nxt, s74)
        s73 = jnp.where((cprev < 15.0) & (c >= 15.0), nxt, s73)
        cprev = c
    thresh = (s73 * _Q) + (s74 * _P)
    cmp01 = jnp.where(e >= thresh, jnp.float32(1.0), jnp.float32(0.0)).astype(
        jnp.bfloat16
    )
    out_ref[0] = jax.lax.dot_general(
        p_ref[...],
        cmp01,
        dimension_numbers=(((1,), (0,)), ((), ())),
        preferred_element_type=jnp.float32,
    )


def kernel(mel, key_bins):
    if mel.ndim == 4:
        mel = mel[:, 0]
    b, m, t = mel.shape
    k = key_bins.shape[0]
    kb = key_bins.astype(jnp.int32)
    u = jnp.unique(kb, size=_D, fill_value=-1)
    pmat = (u[None, :] == kb[:, None]).astype(jnp.bfloat16)
    w = jnp.sum((u[None, :] == kb[:, None]).astype(jnp.float32), axis=0)[:, None]
    g1 = jax.nn.one_hot(u, m, dtype=jnp.float32)
    g2 = jax.nn.one_hot(2 * u, m, dtype=jnp.float32)
    g3 = jax.nn.one_hot(3 * u, m, dtype=jnp.float32)
    gcat = jnp.concatenate([g1, g2, g3], axis=0)
    z2 = jnp.where(2 * u < m, 0.0, 1.0).astype(jnp.float32)[:, None]
    z3 = jnp.where(3 * u < m, 0.0, 1.0).astype(jnp.float32)[:, None]
    tb = 1024
    grid = (b, t // tb)
    out = pl.pallas_call(
        _body,
        grid=grid,
        in_specs=[
            pl.BlockSpec((3 * _D, m), lambda bi, ti: (0, 0)),
            pl.BlockSpec((_D, 1), lambda bi, ti: (0, 0)),
            pl.BlockSpec((_D, 1), lambda bi, ti: (0, 0)),
            pl.BlockSpec((_D, 1), lambda bi, ti: (0, 0)),
            pl.BlockSpec((k, _D), lambda bi, ti: (0, 0)),
            pl.BlockSpec((1, m, tb), lambda bi, ti: (bi, 0, ti)),
        ],
        out_specs=pl.BlockSpec((1, k, tb), lambda bi, ti: (bi, 0, ti)),
        out_shape=jax.ShapeDtypeStruct((b, k, t), jnp.float32),
    )(gcat, z2, z3, w, pmat, mel)
    return (out, out)


# speedup vs baseline: 7.8309x; 1.1133x over previous
import functools

import jax
import jax.numpy as jnp
import numpy as np
from jax.experimental import pallas as pl

_IDX = np.float32(np.float32(np.float32(85.0) / np.float32(100.0)) * np.float32(87.0))
_P = np.float32(_IDX - np.float32(73.0))
_Q = np.float32(np.float32(1.0) - _P)

_NUM_DISTINCT = 15
_D = 64


def _body(gcat_ref, z2_ref, z3_ref, w_ref, p_ref, mel_ref, out_ref):
    spec = jnp.exp(mel_ref[0])
    g = jax.lax.dot_general(
        gcat_ref[...],
        spec,
        dimension_numbers=(((1,), (0,)), ((), ())),
        precision=jax.lax.Precision.HIGHEST,
        preferred_element_type=jnp.float32,
    )
    g1 = g[:_D]
    g2 = g[_D:2 * _D]
    g3 = g[2 * _D:]
    e = (g1 * (g2 + z2_ref[...])) * (g3 + z3_ref[...])

    tcols = e.shape[1]
    neg = jnp.float32(-jnp.inf)
    w = w_ref[...]
    consumed = jnp.zeros(e.shape, jnp.bool_)
    cprev = jnp.zeros((1, tcols), jnp.float32)
    s74 = jnp.zeros((1, tcols), jnp.float32)
    s73 = jnp.zeros((1, tcols), jnp.float32)
    for _ in range(_NUM_DISTINCT):
        masked = jnp.where(consumed, neg, e)
        nxt = jnp.max(masked, axis=0, keepdims=True)
        consumed = e >= nxt
        c = jnp.sum(
            jnp.where(consumed, w, jnp.float32(0.0)),
            axis=0,
            keepdims=True,
        )
        s74 = jnp.where((cprev < 14.0) & (c >= 14.0), nxt, s74)
        s73 = jnp.where((cprev < 15.0) & (c >= 15.0), nxt, s73)
        cprev = c
    thresh = (s73 * _Q) + (s74 * _P)
    cmp01 = jnp.where(e >= thresh, jnp.float32(1.0), jnp.float32(0.0)).astype(
        jnp.bfloat16
    )
    out_ref[0] = jax.lax.dot_general(
        p_ref[...],
        cmp01,
        dimension_numbers=(((1,), (0,)), ((), ())),
        preferred_element_type=jnp.float32,
    )


def kernel(mel, key_bins):
    if mel.ndim == 4:
        mel = mel[:, 0]
    b, m, t = mel.shape
    k = key_bins.shape[0]
    kb = key_bins.astype(jnp.int32)
    u = jnp.unique(kb, size=_D, fill_value=-1)
    pmat = (u[None, :] == kb[:, None]).astype(jnp.bfloat16)
    w = jnp.sum((u[None, :] == kb[:, None]).astype(jnp.float32), axis=0)[:, None]
    g1 = jax.nn.one_hot(u, m, dtype=jnp.float32)
    g2 = jax.nn.one_hot(2 * u, m, dtype=jnp.float32)
    g3 = jax.nn.one_hot(3 * u, m, dtype=jnp.float32)
    gcat = jnp.concatenate([g1, g2, g3], axis=0)
    z2 = jnp.where(2 * u < m, 0.0, 1.0).astype(jnp.float32)[:, None]
    z3 = jnp.where(3 * u < m, 0.0, 1.0).astype(jnp.float32)[:, None]
    tb = 2048
    grid = (b, t // tb)
    out = pl.pallas_call(
        _body,
        grid=grid,
        in_specs=[
            pl.BlockSpec((3 * _D, m), lambda bi, ti: (0, 0)),
            pl.BlockSpec((_D, 1), lambda bi, ti: (0, 0)),
            pl.BlockSpec((_D, 1), lambda bi, ti: (0, 0)),
            pl.BlockSpec((_D, 1), lambda bi, ti: (0, 0)),
            pl.BlockSpec((k, _D), lambda bi, ti: (0, 0)),
            pl.BlockSpec((1, m, tb), lambda bi, ti: (bi, 0, ti)),
        ],
        out_specs=pl.BlockSpec((1, k, tb), lambda bi, ti: (bi, 0, ti)),
        out_shape=jax.ShapeDtypeStruct((b, k, t), jnp.float32),
    )(gcat, z2, z3, w, pmat, mel)
    return (out, out)


# speedup vs baseline: 8.3529x; 1.0667x over previous
import functools

import jax
import jax.numpy as jnp
import numpy as np
from jax.experimental import pallas as pl

_IDX = np.float32(np.float32(np.float32(85.0) / np.float32(100.0)) * np.float32(87.0))
_P = np.float32(_IDX - np.float32(73.0))
_Q = np.float32(np.float32(1.0) - _P)

_NUM_DISTINCT = 15
_D = 64


def _body(gcat_ref, z2_ref, z3_ref, w_ref, p_ref, mel_ref, out_ref):
    spec = jnp.exp(mel_ref[0])
    g = jax.lax.dot_general(
        gcat_ref[...],
        spec,
        dimension_numbers=(((1,), (0,)), ((), ())),
        precision=jax.lax.Precision.HIGHEST,
        preferred_element_type=jnp.float32,
    )
    g1 = g[:_D]
    g2 = g[_D:2 * _D]
    g3 = g[2 * _D:]
    e = (g1 * (g2 + z2_ref[...])) * (g3 + z3_ref[...])

    tcols = e.shape[1]
    neg = jnp.float32(-jnp.inf)
    w = w_ref[...]
    consumed = jnp.zeros(e.shape, jnp.bool_)
    cprev = jnp.zeros((1, tcols), jnp.float32)
    s74 = jnp.zeros((1, tcols), jnp.float32)
    s73 = jnp.zeros((1, tcols), jnp.float32)
    for _ in range(_NUM_DISTINCT):
        masked = jnp.where(consumed, neg, e)
        nxt = jnp.max(masked, axis=0, keepdims=True)
        consumed = e >= nxt
        c = jnp.sum(
            jnp.where(consumed, w, jnp.float32(0.0)),
            axis=0,
            keepdims=True,
        )
        s74 = jnp.where((cprev < 14.0) & (c >= 14.0), nxt, s74)
        s73 = jnp.where((cprev < 15.0) & (c >= 15.0), nxt, s73)
        cprev = c
    thresh = (s73 * _Q) + (s74 * _P)
    cmp01 = jnp.where(e >= thresh, jnp.float32(1.0), jnp.float32(0.0)).astype(
        jnp.bfloat16
    )
    out_ref[0] = jax.lax.dot_general(
        p_ref[...],
        cmp01,
        dimension_numbers=(((1,), (0,)), ((), ())),
        preferred_element_type=jnp.float32,
    )


def kernel(mel, key_bins):
    if mel.ndim == 4:
        mel = mel[:, 0]
    b, m, t = mel.shape
    k = key_bins.shape[0]
    kb = key_bins.astype(jnp.int32)
    u = jnp.unique(kb, size=_D, fill_value=-1)
    pmat = (u[None, :] == kb[:, None]).astype(jnp.bfloat16)
    w = jnp.sum((u[None, :] == kb[:, None]).astype(jnp.float32), axis=0)[:, None]
    g1 = jax.nn.one_hot(u, m, dtype=jnp.float32)
    g2 = jax.nn.one_hot(2 * u, m, dtype=jnp.float32)
    g3 = jax.nn.one_hot(3 * u, m, dtype=jnp.float32)
    gcat = jnp.concatenate([g1, g2, g3], axis=0)
    z2 = jnp.where(2 * u < m, 0.0, 1.0).astype(jnp.float32)[:, None]
    z3 = jnp.where(3 * u < m, 0.0, 1.0).astype(jnp.float32)[:, None]
    tb = 4096
    grid = (b, t // tb)
    out = pl.pallas_call(
        _body,
        grid=grid,
        in_specs=[
            pl.BlockSpec((3 * _D, m), lambda bi, ti: (0, 0)),
            pl.BlockSpec((_D, 1), lambda bi, ti: (0, 0)),
            pl.BlockSpec((_D, 1), lambda bi, ti: (0, 0)),
            pl.BlockSpec((_D, 1), lambda bi, ti: (0, 0)),
            pl.BlockSpec((k, _D), lambda bi, ti: (0, 0)),
            pl.BlockSpec((1, m, tb), lambda bi, ti: (bi, 0, ti)),
        ],
        out_specs=pl.BlockSpec((1, k, tb), lambda bi, ti: (bi, 0, ti)),
        out_shape=jax.ShapeDtypeStruct((b, k, t), jnp.float32),
    )(gcat, z2, z3, w, pmat, mel)
    return (out, out)


# speedup vs baseline: 9.4364x; 1.1297x over previous
import functools

import jax
import jax.numpy as jnp
import numpy as np
from jax.experimental import pallas as pl

_IDX = np.float32(np.float32(np.float32(85.0) / np.float32(100.0)) * np.float32(87.0))
_P = np.float32(_IDX - np.float32(73.0))
_Q = np.float32(np.float32(1.0) - _P)

_NUM_DISTINCT = 15
_D = 64


def _body(gcat_ref, z2_ref, z3_ref, w_ref, p_ref, mel_ref, out_ref):
    spec = jnp.exp(mel_ref[0])
    g = jax.lax.dot_general(
        gcat_ref[...],
        spec,
        dimension_numbers=(((1,), (0,)), ((), ())),
        precision=jax.lax.Precision.HIGHEST,
        preferred_element_type=jnp.float32,
    )
    g1 = g[:_D]
    g2 = g[_D:2 * _D]
    g3 = g[2 * _D:]
    e = (g1 * (g2 + z2_ref[...])) * (g3 + z3_ref[...])

    tcols = e.shape[1]
    neg = jnp.float32(-jnp.inf)
    wrow = w_ref[...]

    def count(mask):
        ind = jnp.where(mask, jnp.float32(1.0), jnp.float32(0.0))
        return jax.lax.dot_general(
            wrow,
            ind,
            dimension_numbers=(((1,), (0,)), ((), ())),
            preferred_element_type=jnp.float32,
        )

    nxt = jnp.max(e, axis=0, keepdims=True)
    consumed = e >= nxt
    c = count(consumed)
    s74 = jnp.where(c >= 14.0, nxt, jnp.float32(0.0))
    s73 = jnp.where(c >= 15.0, nxt, jnp.float32(0.0))
    cprev = c
    for _ in range(_NUM_DISTINCT - 1):
        masked = jnp.where(consumed, neg, e)
        nxt = jnp.max(masked, axis=0, keepdims=True)
        consumed = e >= nxt
        c = count(consumed)
        s74 = jnp.where((cprev < 14.0) & (c >= 14.0), nxt, s74)
        s73 = jnp.where((cprev < 15.0) & (c >= 15.0), nxt, s73)
        cprev = c
    thresh = (s73 * _Q) + (s74 * _P)
    cmp01 = jnp.where(e >= thresh, jnp.float32(1.0), jnp.float32(0.0)).astype(
        jnp.bfloat16
    )
    out_ref[0] = jax.lax.dot_general(
        p_ref[...],
        cmp01,
        dimension_numbers=(((1,), (0,)), ((), ())),
        preferred_element_type=jnp.float32,
    )


def kernel(mel, key_bins):
    if mel.ndim == 4:
        mel = mel[:, 0]
    b, m, t = mel.shape
    k = key_bins.shape[0]
    kb = key_bins.astype(jnp.int32)
    u = jnp.unique(kb, size=_D, fill_value=-1)
    pmat = (u[None, :] == kb[:, None]).astype(jnp.bfloat16)
    w = jnp.sum((u[None, :] == kb[:, None]).astype(jnp.float32), axis=0)[None, :]
    g1 = jax.nn.one_hot(u, m, dtype=jnp.float32)
    g2 = jax.nn.one_hot(2 * u, m, dtype=jnp.float32)
    g3 = jax.nn.one_hot(3 * u, m, dtype=jnp.float32)
    gcat = jnp.concatenate([g1, g2, g3], axis=0)
    z2 = jnp.where(2 * u < m, 0.0, 1.0).astype(jnp.float32)[:, None]
    z3 = jnp.where(3 * u < m, 0.0, 1.0).astype(jnp.float32)[:, None]
    tb = 4096
    grid = (b, t // tb)
    out = pl.pallas_call(
        _body,
        grid=grid,
        in_specs=[
            pl.BlockSpec((3 * _D, m), lambda bi, ti: (0, 0)),
            pl.BlockSpec((_D, 1), lambda bi, ti: (0, 0)),
            pl.BlockSpec((_D, 1), lambda bi, ti: (0, 0)),
            pl.BlockSpec((1, _D), lambda bi, ti: (0, 0)),
            pl.BlockSpec((k, _D), lambda bi, ti: (0, 0)),
            pl.BlockSpec((1, m, tb), lambda bi, ti: (bi, 0, ti)),
        ],
        out_specs=pl.BlockSpec((1, k, tb), lambda bi, ti: (bi, 0, ti)),
        out_shape=jax.ShapeDtypeStruct((b, k, t), jnp.float32),
    )(gcat, z2, z3, w, pmat, mel)
    return (out, out)
